# baseline (speedup 1.0000x reference)
import functools

import jax
import jax.numpy as jnp
from jax import lax
from jax.experimental import pallas as pl
from jax.experimental.pallas import tpu as pltpu
from jax.experimental.pallas import tpu_sc as plsc

N = 10000
E = 320000
D = 128
H = 128

NC = 2
NS = 16
NW = NC * NS
NPW = 320
NPAD = NW * NPW
C = 2000
NCHUNK = E // C
CG = C // 16
K = 128
MBUF = 2048 + 16
NEG = float("-inf")


def _scatter_max_body(edge_hbm, msg_hbm, fwd_hbm, bwd_hbm,
                      acc, row_buf, col_buf, meid, moff, rows, sem_g, sem_i):
    wid = lax.axis_index("s") * NC + lax.axis_index("c")
    lo = wid * NPW
    iota = lax.iota(jnp.int32, 16)

    def init_acc(r, _):
        for t in range(8):
            acc[r, pl.ds(t * 16, 16)] = jnp.full((16,), NEG, jnp.float32)
        return 0
    lax.fori_loop(0, 2 * NPW, init_acc, 0)

    def init_match(i, _):
        z = jnp.zeros((16,), jnp.int32)
        meid[pl.ds(i * 16, 16)] = z
        moff[pl.ds(i * 16, 16)] = z
        return 0
    lax.fori_loop(0, MBUF // 16, init_match, 0)

    def chunk_body(c_i, _):
        pltpu.sync_copy(edge_hbm.at[pl.ds(c_i * C, C)], row_buf)
        pltpu.sync_copy(edge_hbm.at[pl.ds(E + c_i * C, C)], col_buf)
        ebase = c_i * C

        for agg in range(2):
            buf = col_buf if agg == 0 else row_buf
            abase = agg * NPW

            def fbody(i, cnt, buf=buf):
                v = buf[pl.ds(i * 16, 16)]
                off = v - lo
                m = (off >= 0) & (off < NPW)
                mi = jnp.where(m, 1, 0)
                cs = plsc.cumsum(mi)
                pos = (cnt + cs) - mi
                eidv = iota + (ebase + i * 16)
                plsc.store_scatter(meid, [pos], eidv, mask=m)
                plsc.store_scatter(moff, [pos], off, mask=m)
                return cnt + cs[15]
            mcnt = lax.fori_loop(0, CG, fbody, 0)

            nb = (mcnt + (K - 1)) // K

            def bbody(b, _, abase=abase):
                pltpu.async_copy(
                    msg_hbm.at[meid.at[pl.ds(b * K, K)]], rows, sem_g).wait()
                jmax = jnp.minimum(K, mcnt - b * K)

                def jbody(j, _):
                    o = abase + moff[pl.ds(b * K + j, 16)][0]
                    for t in range(8):
                        sl = pl.ds(t * 16, 16)
                        acc[o, sl] = jnp.maximum(acc[o, sl], rows[j, sl])
                    return 0
                lax.fori_loop(0, jmax, jbody, 0)
                return 0
            lax.fori_loop(0, nb, bbody, 0)
        return 0
    lax.fori_loop(0, NCHUNK, chunk_body, 0)

    def fix(r, _):
        for t in range(8):
            sl = pl.ds(t * 16, 16)
            v = acc[r, sl]
            acc[r, sl] = jnp.where(v == NEG, 0.0, v)
        return 0
    lax.fori_loop(0, 2 * NPW, fix, 0)

    pltpu.sync_copy(acc.at[pl.ds(0, NPW)], fwd_hbm.at[pl.ds(lo, NPW)])
    pltpu.sync_copy(acc.at[pl.ds(NPW, NPW)], bwd_hbm.at[pl.ds(lo, NPW)])


_scatter_max = functools.partial(
    pl.kernel,
    out_type=(jax.ShapeDtypeStruct((NPAD, H), jnp.float32),
              jax.ShapeDtypeStruct((NPAD, H), jnp.float32)),
    mesh=plsc.VectorSubcoreMesh(core_axis_name="c", subcore_axis_name="s"),
    scratch_types=[
        pltpu.VMEM((2 * NPW, H), jnp.float32),
        pltpu.VMEM((C,), jnp.int32),
        pltpu.VMEM((C,), jnp.int32),
        pltpu.VMEM((MBUF,), jnp.int32),
        pltpu.VMEM((MBUF,), jnp.int32),
        pltpu.VMEM((K, H), jnp.float32),
        pltpu.SemaphoreType.DMA,
        pltpu.SemaphoreType.DMA,
    ],
    compiler_params=pltpu.CompilerParams(needs_layout_passes=False),
)(_scatter_max_body)


BLK = 1000
GRID = N // BLK
EPS = 1e-5


def _stage1_body(x_ref, f_ref, b_ref, W1_ref, b1_ref, y_ref, ss_ref, sq_ref):
    i = pl.program_id(0)
    y = (jnp.dot(x_ref[...], W1_ref[0:D, :], preferred_element_type=jnp.float32)
         + jnp.dot(f_ref[...], W1_ref[D:D + H, :], preferred_element_type=jnp.float32)
         + jnp.dot(b_ref[...], W1_ref[D + H:, :], preferred_element_type=jnp.float32)
         + b1_ref[...])
    y_ref[...] = y

    @pl.when(i == 0)
    def _():
        ss_ref[...] = jnp.zeros_like(ss_ref)
        sq_ref[...] = jnp.zeros_like(sq_ref)
    ss_ref[...] += jnp.sum(y, axis=0, keepdims=True)
    sq_ref[...] += jnp.sum(y * y, axis=0, keepdims=True)


def _stage2_body(y_ref, ss_ref, sq_ref, g_ref, be_ref, W2_ref, b2_ref,
                 y2_ref, ss2_ref, sq2_ref):
    i = pl.program_id(0)
    mean = ss_ref[...] / N
    var = sq_ref[...] / N - mean * mean
    inv = lax.rsqrt(var + EPS)
    h1 = jax.nn.relu((y_ref[...] - mean) * inv * g_ref[...] + be_ref[...])
    y2 = jnp.dot(h1, W2_ref[...], preferred_element_type=jnp.float32) + b2_ref[...]
    y2_ref[...] = y2

    @pl.when(i == 0)
    def _():
        ss2_ref[...] = jnp.zeros_like(ss2_ref)
        sq2_ref[...] = jnp.zeros_like(sq2_ref)
    ss2_ref[...] += jnp.sum(y2, axis=0, keepdims=True)
    sq2_ref[...] += jnp.sum(y2 * y2, axis=0, keepdims=True)


def _stage3_body(y2_ref, ss2_ref, sq2_ref, g_ref, be_ref, wa_ref, ba_ref,
                 h_ref, att_ref):
    mean = ss2_ref[...] / N
    var = sq2_ref[...] / N - mean * mean
    inv = lax.rsqrt(var + EPS)
    h2 = jax.nn.relu((y2_ref[...] - mean) * inv * g_ref[...] + be_ref[...])
    h_ref[...] = h2
    logit = jnp.sum(h2 * wa_ref[...], axis=1, keepdims=True) + ba_ref[0, 0]
    att_ref[...] = jnp.broadcast_to(jax.nn.sigmoid(logit), (BLK, 8))


def _row_spec(shape):
    return pl.BlockSpec(shape, lambda i: (0, 0))


def _blk_spec(w):
    return pl.BlockSpec((BLK, w), lambda i: (i, 0))


def _mlp(x, fwd, bwd, W1, b1, gamma1, beta1, W2, b2, gamma2, beta2, Wa, ba):
    b1r = b1.reshape(1, H)
    g1r = gamma1.reshape(1, H)
    be1r = beta1.reshape(1, H)
    b2r = b2.reshape(1, H)
    g2r = gamma2.reshape(1, H)
    be2r = beta2.reshape(1, H)
    war = Wa.reshape(1, H)
    bar = jnp.broadcast_to(ba.reshape(1, 1), (1, H))

    y1, ss1, sq1 = pl.pallas_call(
        _stage1_body,
        grid=(GRID,),
        in_specs=[_blk_spec(D), _blk_spec(H), _blk_spec(H),
                  _row_spec((D + 2 * H, H)), _row_spec((1, H))],
        out_specs=[_blk_spec(H), _row_spec((1, H)), _row_spec((1, H))],
        out_shape=[jax.ShapeDtypeStruct((N, H), jnp.float32),
                   jax.ShapeDtypeStruct((1, H), jnp.float32),
                   jax.ShapeDtypeStruct((1, H), jnp.float32)],
    )(x, fwd, bwd, W1, b1r)

    y2, ss2, sq2 = pl.pallas_call(
        _stage2_body,
        grid=(GRID,),
        in_specs=[_blk_spec(H), _row_spec((1, H)), _row_spec((1, H)),
                  _row_spec((1, H)), _row_spec((1, H)),
                  _row_spec((H, H)), _row_spec((1, H))],
        out_specs=[_blk_spec(H), _row_spec((1, H)), _row_spec((1, H))],
        out_shape=[jax.ShapeDtypeStruct((N, H), jnp.float32),
                   jax.ShapeDtypeStruct((1, H), jnp.float32),
                   jax.ShapeDtypeStruct((1, H), jnp.float32)],
    )(y1, ss1, sq1, g1r, be1r, W2, b2r)

    h, att8 = pl.pallas_call(
        _stage3_body,
        grid=(GRID,),
        in_specs=[_blk_spec(H), _row_spec((1, H)), _row_spec((1, H)),
                  _row_spec((1, H)), _row_spec((1, H)),
                  _row_spec((1, H)), _row_spec((1, H))],
        out_specs=[_blk_spec(H), _blk_spec(8)],
        out_shape=[jax.ShapeDtypeStruct((N, H), jnp.float32),
                   jax.ShapeDtypeStruct((N, 8), jnp.float32)],
    )(y2, ss2, sq2, g2r, be2r, war, bar)

    return h, att8[:, 0]


def kernel(x, edge_index, message, W1, b1, gamma1, beta1, W2, b2, gamma2, beta2, Wa, ba):
    fwd_p, bwd_p = _scatter_max(edge_index.reshape(-1), message)
    fwd = fwd_p[:N]
    bwd = bwd_p[:N]
    return _mlp(x, fwd, bwd, W1, b1, gamma1, beta1, W2, b2,
                gamma2, beta2, Wa, ba)


# speedup vs baseline: 7.5232x; 7.5232x over previous
import functools

import jax
import jax.numpy as jnp
from jax import lax
from jax.experimental import pallas as pl
from jax.experimental.pallas import tpu as pltpu
from jax.experimental.pallas import tpu_sc as plsc

N = 10000
E = 320000
D = 128
H = 128

NC = 2
NS = 16
NPAD = 10240
CH = 3200
NCHUNK = E // CH
NEG = float("-inf")


def _vperm(v, perm):
    return lax.gather(
        v, perm[:, None],
        lax.GatherDimensionNumbers(offset_dims=(), collapsed_slice_dims=(0,),
                                   start_index_map=(0,)),
        (1,), mode=lax.GatherScatterMode.PROMISE_IN_BOUNDS)


def _scatter_max_body(edge_hbm, msgT_hbm, fwd_hbm, bwd_hbm,
                      acc, idxb, mbuf):
    cid = lax.axis_index("c")
    sid = lax.axis_index("s")
    f0 = sid * 8
    iota = lax.iota(jnp.int32, 16)
    feat = iota & 7
    half = iota >> 3
    xor8 = iota ^ 8
    lane_lt8 = iota < 8
    accbase = feat * NPAD
    idx_off = (1 - cid) * E

    def ia(i, _):
        for t in range(8):
            acc[pl.ds(i * 128 + t * 16, 16)] = jnp.full((16,), NEG, jnp.float32)
        return 0
    lax.fori_loop(0, (8 * NPAD) // 128, ia, 0)

    def chunk(ci, _):
        pltpu.sync_copy(edge_hbm.at[pl.ds(idx_off + ci * CH, CH)], idxb)
        pltpu.sync_copy(msgT_hbm.at[pl.ds(f0, 8), pl.ds(ci * CH, CH)], mbuf)

        def grp(g, _):
            nodevec = idxb[pl.ds(g * 16, 16)]
            ebase = g * 16
            for p in range(8):
                n0 = nodevec[2 * p]
                n1 = nodevec[2 * p + 1]
                npair = jnp.where(lane_lt8, n0, n1)
                mvec = plsc.load_gather(mbuf, [feat, half + (ebase + 2 * p)])
                eq = n0 == n1
                comb = jnp.maximum(mvec, _vperm(mvec, xor8))
                val = jnp.where(eq, comb, mvec)
                wmask = jnp.logical_or(lane_lt8, jnp.logical_not(eq))
                aidx = accbase + npair
                cur = plsc.load_gather(acc, [aidx])
                plsc.store_scatter(acc, [aidx], jnp.maximum(cur, val),
                                   mask=wmask)
            return 0
        lax.fori_loop(0, CH // 16, grp, 0)
        return 0
    lax.fori_loop(0, NCHUNK, chunk, 0)

    def fix(i, _):
        for t in range(8):
            sl = pl.ds(i * 128 + t * 16, 16)
            v = acc[sl]
            acc[sl] = jnp.where(v == NEG, 0.0, v)
        return 0
    lax.fori_loop(0, (8 * NPAD) // 128, fix, 0)

    @pl.when(cid == 0)
    def _():
        pltpu.sync_copy(acc, fwd_hbm.at[pl.ds(f0 * NPAD, 8 * NPAD)])

    @pl.when(cid == 1)
    def _():
        pltpu.sync_copy(acc, bwd_hbm.at[pl.ds(f0 * NPAD, 8 * NPAD)])


_scatter_max = functools.partial(
    pl.kernel,
    out_type=(jax.ShapeDtypeStruct((128 * NPAD,), jnp.float32),
              jax.ShapeDtypeStruct((128 * NPAD,), jnp.float32)),
    mesh=plsc.VectorSubcoreMesh(core_axis_name="c", subcore_axis_name="s"),
    scratch_types=[
        pltpu.VMEM((8 * NPAD,), jnp.float32),
        pltpu.VMEM((CH,), jnp.int32),
        pltpu.VMEM((8, CH), jnp.float32),
    ],
    compiler_params=pltpu.CompilerParams(needs_layout_passes=False),
)(_scatter_max_body)


BLK = 1000
GRID = N // BLK
EPS = 1e-5


def _stage1_body(x_ref, f_ref, b_ref, W1_ref, b1_ref, y_ref, ss_ref, sq_ref):
    i = pl.program_id(0)
    y = (jnp.dot(x_ref[...], W1_ref[0:D, :], preferred_element_type=jnp.float32)
         + jnp.dot(f_ref[...], W1_ref[D:D + H, :], preferred_element_type=jnp.float32)
         + jnp.dot(b_ref[...], W1_ref[D + H:, :], preferred_element_type=jnp.float32)
         + b1_ref[...])
    y_ref[...] = y

    @pl.when(i == 0)
    def _():
        ss_ref[...] = jnp.zeros_like(ss_ref)
        sq_ref[...] = jnp.zeros_like(sq_ref)
    ss_ref[...] += jnp.sum(y, axis=0, keepdims=True)
    sq_ref[...] += jnp.sum(y * y, axis=0, keepdims=True)


def _stage2_body(y_ref, ss_ref, sq_ref, g_ref, be_ref, W2_ref, b2_ref,
                 y2_ref, ss2_ref, sq2_ref):
    i = pl.program_id(0)
    mean = ss_ref[...] / N
    var = sq_ref[...] / N - mean * mean
    inv = lax.rsqrt(var + EPS)
    h1 = jax.nn.relu((y_ref[...] - mean) * inv * g_ref[...] + be_ref[...])
    y2 = jnp.dot(h1, W2_ref[...], preferred_element_type=jnp.float32) + b2_ref[...]
    y2_ref[...] = y2

    @pl.when(i == 0)
    def _():
        ss2_ref[...] = jnp.zeros_like(ss2_ref)
        sq2_ref[...] = jnp.zeros_like(sq2_ref)
    ss2_ref[...] += jnp.sum(y2, axis=0, keepdims=True)
    sq2_ref[...] += jnp.sum(y2 * y2, axis=0, keepdims=True)


def _stage3_body(y2_ref, ss2_ref, sq2_ref, g_ref, be_ref, wa_ref, ba_ref,
                 h_ref, att_ref):
    mean = ss2_ref[...] / N
    var = sq2_ref[...] / N - mean * mean
    inv = lax.rsqrt(var + EPS)
    h2 = jax.nn.relu((y2_ref[...] - mean) * inv * g_ref[...] + be_ref[...])
    h_ref[...] = h2
    logit = jnp.sum(h2 * wa_ref[...], axis=1, keepdims=True) + ba_ref[0, 0]
    att_ref[...] = jnp.broadcast_to(jax.nn.sigmoid(logit), (BLK, 8))


def _row_spec(shape):
    return pl.BlockSpec(shape, lambda i: (0, 0))


def _blk_spec(w):
    return pl.BlockSpec((BLK, w), lambda i: (i, 0))


def _mlp(x, fwd, bwd, W1, b1, gamma1, beta1, W2, b2, gamma2, beta2, Wa, ba):
    b1r = b1.reshape(1, H)
    g1r = gamma1.reshape(1, H)
    be1r = beta1.reshape(1, H)
    b2r = b2.reshape(1, H)
    g2r = gamma2.reshape(1, H)
    be2r = beta2.reshape(1, H)
    war = Wa.reshape(1, H)
    bar = jnp.broadcast_to(ba.reshape(1, 1), (1, H))

    y1, ss1, sq1 = pl.pallas_call(
        _stage1_body,
        grid=(GRID,),
        in_specs=[_blk_spec(D), _blk_spec(H), _blk_spec(H),
                  _row_spec((D + 2 * H, H)), _row_spec((1, H))],
        out_specs=[_blk_spec(H), _row_spec((1, H)), _row_spec((1, H))],
        out_shape=[jax.ShapeDtypeStruct((N, H), jnp.float32),
                   jax.ShapeDtypeStruct((1, H), jnp.float32),
                   jax.ShapeDtypeStruct((1, H), jnp.float32)],
    )(x, fwd, bwd, W1, b1r)

    y2, ss2, sq2 = pl.pallas_call(
        _stage2_body,
        grid=(GRID,),
        in_specs=[_blk_spec(H), _row_spec((1, H)), _row_spec((1, H)),
                  _row_spec((1, H)), _row_spec((1, H)),
                  _row_spec((H, H)), _row_spec((1, H))],
        out_specs=[_blk_spec(H), _row_spec((1, H)), _row_spec((1, H))],
        out_shape=[jax.ShapeDtypeStruct((N, H), jnp.float32),
                   jax.ShapeDtypeStruct((1, H), jnp.float32),
                   jax.ShapeDtypeStruct((1, H), jnp.float32)],
    )(y1, ss1, sq1, g1r, be1r, W2, b2r)

    h, att8 = pl.pallas_call(
        _stage3_body,
        grid=(GRID,),
        in_specs=[_blk_spec(H), _row_spec((1, H)), _row_spec((1, H)),
                  _row_spec((1, H)), _row_spec((1, H)),
                  _row_spec((1, H)), _row_spec((1, H))],
        out_specs=[_blk_spec(H), _blk_spec(8)],
        out_shape=[jax.ShapeDtypeStruct((N, H), jnp.float32),
                   jax.ShapeDtypeStruct((N, 8), jnp.float32)],
    )(y2, ss2, sq2, g2r, be2r, war, bar)

    return h, att8[:, 0]


def kernel(x, edge_index, message, W1, b1, gamma1, beta1, W2, b2, gamma2, beta2, Wa, ba):
    edge_flat = edge_index.reshape(-1)
    msgT = message.T
    fwd_f, bwd_f = _scatter_max(edge_flat, msgT)
    fwd = fwd_f.reshape(128, NPAD).T[:N]
    bwd = bwd_f.reshape(128, NPAD).T[:N]
    return _mlp(x, fwd, bwd, W1, b1, gamma1, beta1, W2, b2,
                gamma2, beta2, Wa, ba)


# speedup vs baseline: 7.8939x; 1.0493x over previous
import functools

import jax
import jax.numpy as jnp
from jax import lax
from jax.experimental import pallas as pl
from jax.experimental.pallas import tpu as pltpu
from jax.experimental.pallas import tpu_sc as plsc

N = 10000
E = 320000
D = 128
H = 128

NC = 2
NS = 16
NPAD = 10240
CH = 3200
NCHUNK = E // CH
NEG = float("-inf")


def _vperm(v, perm):
    return lax.gather(
        v, perm[:, None],
        lax.GatherDimensionNumbers(offset_dims=(), collapsed_slice_dims=(0,),
                                   start_index_map=(0,)),
        (1,), mode=lax.GatherScatterMode.PROMISE_IN_BOUNDS)


def _scatter_max_body(edge_hbm, msgT_hbm, fwd_hbm, bwd_hbm,
                      acc, idxb, mbuf):
    cid = lax.axis_index("c")
    sid = lax.axis_index("s")
    f0 = sid * 8
    iota = lax.iota(jnp.int32, 16)
    feat = iota & 7
    half = iota >> 3
    xor8 = iota ^ 8
    lane_lt8 = iota < 8
    accbase = feat * NPAD
    idx_off = (1 - cid) * E

    def ia(i, _):
        for t in range(8):
            acc[pl.ds(i * 128 + t * 16, 16)] = jnp.full((16,), NEG, jnp.float32)
        return 0
    lax.fori_loop(0, (8 * NPAD) // 128, ia, 0)

    def chunk(ci, _):
        pltpu.sync_copy(edge_hbm.at[pl.ds(idx_off + ci * CH, CH)], idxb)
        pltpu.sync_copy(msgT_hbm.at[pl.ds(f0, 8), pl.ds(ci * CH, CH)], mbuf)

        def grp(g, _):
            nodevec = idxb[pl.ds(g * 16, 16)]
            ebase = g * 16
            for p in range(8):
                sel = half + 2 * p
                npair = _vperm(nodevec, sel)
                npx8 = _vperm(nodevec, sel ^ 1)
                eqv = npair == npx8
                mvec = plsc.load_gather(mbuf, [feat, half + (ebase + 2 * p)])
                comb = jnp.maximum(mvec, _vperm(mvec, xor8))
                val = jnp.where(eqv, comb, mvec)
                wmask = jnp.logical_or(lane_lt8, jnp.logical_not(eqv))
                aidx = accbase + npair
                cur = plsc.load_gather(acc, [aidx])
                plsc.store_scatter(acc, [aidx], jnp.maximum(cur, val),
                                   mask=wmask)
            return 0
        lax.fori_loop(0, CH // 16, grp, 0)
        return 0
    lax.fori_loop(0, NCHUNK, chunk, 0)

    def fix(i, _):
        for t in range(8):
            sl = pl.ds(i * 128 + t * 16, 16)
            v = acc[sl]
            acc[sl] = jnp.where(v == NEG, 0.0, v)
        return 0
    lax.fori_loop(0, (8 * NPAD) // 128, fix, 0)

    @pl.when(cid == 0)
    def _():
        pltpu.sync_copy(acc, fwd_hbm.at[pl.ds(f0 * NPAD, 8 * NPAD)])

    @pl.when(cid == 1)
    def _():
        pltpu.sync_copy(acc, bwd_hbm.at[pl.ds(f0 * NPAD, 8 * NPAD)])


_scatter_max = functools.partial(
    pl.kernel,
    out_type=(jax.ShapeDtypeStruct((128 * NPAD,), jnp.float32),
              jax.ShapeDtypeStruct((128 * NPAD,), jnp.float32)),
    mesh=plsc.VectorSubcoreMesh(core_axis_name="c", subcore_axis_name="s"),
    scratch_types=[
        pltpu.VMEM((8 * NPAD,), jnp.float32),
        pltpu.VMEM((CH,), jnp.int32),
        pltpu.VMEM((8, CH), jnp.float32),
    ],
    compiler_params=pltpu.CompilerParams(needs_layout_passes=False),
)(_scatter_max_body)


BLK = 1000
GRID = N // BLK
EPS = 1e-5


def _stage1_body(x_ref, f_ref, b_ref, W1_ref, b1_ref, y_ref, ss_ref, sq_ref):
    i = pl.program_id(0)
    y = (jnp.dot(x_ref[...], W1_ref[0:D, :], preferred_element_type=jnp.float32)
         + jnp.dot(f_ref[...], W1_ref[D:D + H, :], preferred_element_type=jnp.float32)
         + jnp.dot(b_ref[...], W1_ref[D + H:, :], preferred_element_type=jnp.float32)
         + b1_ref[...])
    y_ref[...] = y

    @pl.when(i == 0)
    def _():
        ss_ref[...] = jnp.zeros_like(ss_ref)
        sq_ref[...] = jnp.zeros_like(sq_ref)
    ss_ref[...] += jnp.sum(y, axis=0, keepdims=True)
    sq_ref[...] += jnp.sum(y * y, axis=0, keepdims=True)


def _stage2_body(y_ref, ss_ref, sq_ref, g_ref, be_ref, W2_ref, b2_ref,
                 y2_ref, ss2_ref, sq2_ref):
    i = pl.program_id(0)
    mean = ss_ref[...] / N
    var = sq_ref[...] / N - mean * mean
    inv = lax.rsqrt(var + EPS)
    h1 = jax.nn.relu((y_ref[...] - mean) * inv * g_ref[...] + be_ref[...])
    y2 = jnp.dot(h1, W2_ref[...], preferred_element_type=jnp.float32) + b2_ref[...]
    y2_ref[...] = y2

    @pl.when(i == 0)
    def _():
        ss2_ref[...] = jnp.zeros_like(ss2_ref)
        sq2_ref[...] = jnp.zeros_like(sq2_ref)
    ss2_ref[...] += jnp.sum(y2, axis=0, keepdims=True)
    sq2_ref[...] += jnp.sum(y2 * y2, axis=0, keepdims=True)


def _stage3_body(y2_ref, ss2_ref, sq2_ref, g_ref, be_ref, wa_ref, ba_ref,
                 h_ref, att_ref):
    mean = ss2_ref[...] / N
    var = sq2_ref[...] / N - mean * mean
    inv = lax.rsqrt(var + EPS)
    h2 = jax.nn.relu((y2_ref[...] - mean) * inv * g_ref[...] + be_ref[...])
    h_ref[...] = h2
    logit = jnp.sum(h2 * wa_ref[...], axis=1, keepdims=True) + ba_ref[0, 0]
    att_ref[...] = jnp.broadcast_to(jax.nn.sigmoid(logit), (BLK, 8))


def _row_spec(shape):
    return pl.BlockSpec(shape, lambda i: (0, 0))


def _blk_spec(w):
    return pl.BlockSpec((BLK, w), lambda i: (i, 0))


def _mlp(x, fwd, bwd, W1, b1, gamma1, beta1, W2, b2, gamma2, beta2, Wa, ba):
    b1r = b1.reshape(1, H)
    g1r = gamma1.reshape(1, H)
    be1r = beta1.reshape(1, H)
    b2r = b2.reshape(1, H)
    g2r = gamma2.reshape(1, H)
    be2r = beta2.reshape(1, H)
    war = Wa.reshape(1, H)
    bar = jnp.broadcast_to(ba.reshape(1, 1), (1, H))

    y1, ss1, sq1 = pl.pallas_call(
        _stage1_body,
        grid=(GRID,),
        in_specs=[_blk_spec(D), _blk_spec(H), _blk_spec(H),
                  _row_spec((D + 2 * H, H)), _row_spec((1, H))],
        out_specs=[_blk_spec(H), _row_spec((1, H)), _row_spec((1, H))],
        out_shape=[jax.ShapeDtypeStruct((N, H), jnp.float32),
                   jax.ShapeDtypeStruct((1, H), jnp.float32),
                   jax.ShapeDtypeStruct((1, H), jnp.float32)],
    )(x, fwd, bwd, W1, b1r)

    y2, ss2, sq2 = pl.pallas_call(
        _stage2_body,
        grid=(GRID,),
        in_specs=[_blk_spec(H), _row_spec((1, H)), _row_spec((1, H)),
                  _row_spec((1, H)), _row_spec((1, H)),
                  _row_spec((H, H)), _row_spec((1, H))],
        out_specs=[_blk_spec(H), _row_spec((1, H)), _row_spec((1, H))],
        out_shape=[jax.ShapeDtypeStruct((N, H), jnp.float32),
                   jax.ShapeDtypeStruct((1, H), jnp.float32),
                   jax.ShapeDtypeStruct((1, H), jnp.float32)],
    )(y1, ss1, sq1, g1r, be1r, W2, b2r)

    h, att8 = pl.pallas_call(
        _stage3_body,
        grid=(GRID,),
        in_specs=[_blk_spec(H), _row_spec((1, H)), _row_spec((1, H)),
                  _row_spec((1, H)), _row_spec((1, H)),
                  _row_spec((1, H)), _row_spec((1, H))],
        out_specs=[_blk_spec(H), _blk_spec(8)],
        out_shape=[jax.ShapeDtypeStruct((N, H), jnp.float32),
                   jax.ShapeDtypeStruct((N, 8), jnp.float32)],
    )(y2, ss2, sq2, g2r, be2r, war, bar)

    return h, att8[:, 0]


def kernel(x, edge_index, message, W1, b1, gamma1, beta1, W2, b2, gamma2, beta2, Wa, ba):
    edge_flat = edge_index.reshape(-1)
    msgT = message.T
    fwd_f, bwd_f = _scatter_max(edge_flat, msgT)
    fwd = fwd_f.reshape(128, NPAD).T[:N]
    bwd = bwd_f.reshape(128, NPAD).T[:N]
    return _mlp(x, fwd, bwd, W1, b1, gamma1, beta1, W2, b2,
                gamma2, beta2, Wa, ba)
